# baseline (speedup 1.0000x reference)
import functools

import jax
import jax.numpy as jnp
from jax import lax
from jax.experimental import pallas as pl
from jax.experimental.pallas import tpu as pltpu
from jax.experimental.pallas import tpu_sc as plsc

NUM_BINS = 8
INV_BIN = 8.0
NUM_HEADS = 16
SIDE = 2 * NUM_BINS + 1
TABLE_SIZE = SIDE ** 3
TPAD = 4992

NC, NS, L = 2, 16, 16
NW = NC * NS


def _splat(x):
    return jnp.broadcast_to(jnp.asarray(x, jnp.int32), (L,))


def _body(coords_hbm, table_hbm, out_hbm, coords_v, table_v, out_v,
          sem0, sem1, sem2, sem3, B, N):
    rows_per_w = (B * N) // NW
    tiles_per_b = N // rows_per_w
    wid = lax.axis_index("s") * NC + lax.axis_index("c")
    b = wid // tiles_per_b
    ibase = (wid % tiles_per_b) * rows_per_w

    pltpu.sync_copy(table_hbm, table_v)
    pltpu.sync_copy(coords_hbm.at[b], coords_v)

    sems = (sem0, sem1, sem2, sem3)
    n_chunks = N // L

    def splat_center(i):
        si = jnp.broadcast_to(i, (L,))
        return [plsc.load_gather(coords_v, [_splat(c), si]) + 8.5
                for c in range(3)]

    def do_pair(t2, _):
        r0 = t2 * 2
        iA, iB = ibase + r0, ibase + r0 + 1
        cA = splat_center(iA)
        cB = splat_center(iB)

        for half in range(2):
            j0 = half * (N // 2)
            mA, mB = half * 2, half * 2 + 1

            @pl.when(t2 > 0)
            def _():
                for m in (mA, mB):
                    pltpu.make_async_copy(
                        out_v.at[m], out_hbm.at[0, :, 0, pl.ds(0, N // 2)],
                        sems[m]).wait()

            @plsc.parallel_loop(0, n_chunks // 2, unroll=1)
            def chunk(jc):
                jb = jc * L
                cj = [coords_v[c, pl.ds(j0 + jb, L)] for c in range(3)]
                for (m, ci) in ((mA, cA), (mB, cB)):
                    q = [jnp.clip(ci[c] - cj[c], 0.01, 16.99)
                         for c in range(3)]
                    idx = (q[0].astype(jnp.int32) * (SIDE * SIDE)
                           + q[1].astype(jnp.int32) * SIDE
                           + q[2].astype(jnp.int32))
                    for h in range(NUM_HEADS):
                        val = plsc.load_gather(
                            table_v.at[pl.ds(h * TPAD, TPAD)], [idx])
                        out_v[m, h, pl.ds(jb, L)] = val

            pltpu.async_copy(
                out_v.at[mA], out_hbm.at[b, :, iA, pl.ds(j0, N // 2)],
                sems[mA])
            pltpu.async_copy(
                out_v.at[mB], out_hbm.at[b, :, iB, pl.ds(j0, N // 2)],
                sems[mB])
        return 0

    lax.fori_loop(0, rows_per_w // 2, do_pair, 0)
    for m in range(4):
        pltpu.make_async_copy(
            out_v.at[m], out_hbm.at[0, :, 0, pl.ds(0, N // 2)],
            sems[m]).wait()


def kernel(token_centers, bias_table):
    B, N, _ = token_centers.shape
    H = bias_table.shape[0]
    assert H == NUM_HEADS and bias_table.shape[1] == TABLE_SIZE
    assert (B * N) % (2 * NW) == 0 and N % (4 * L) == 0

    coords = jnp.transpose(token_centers, (0, 2, 1)) * INV_BIN
    table_flat = jnp.pad(bias_table, ((0, 0), (0, TPAD - TABLE_SIZE))).reshape(-1)

    mesh = plsc.VectorSubcoreMesh(
        core_axis_name="c", subcore_axis_name="s",
        num_cores=NC, num_subcores=NS)
    body = functools.partial(_body, B=B, N=N)
    f = pl.kernel(
        body,
        out_type=jax.ShapeDtypeStruct((B, H, N, N), jnp.float32),
        mesh=mesh,
        compiler_params=pltpu.CompilerParams(
            needs_layout_passes=False),
        scratch_types=[
            pltpu.VMEM((3, N), jnp.float32),
            pltpu.VMEM((H * TPAD,), jnp.float32),
            pltpu.VMEM((4, H, N // 2), jnp.float32),
            pltpu.SemaphoreType.DMA,
            pltpu.SemaphoreType.DMA,
            pltpu.SemaphoreType.DMA,
            pltpu.SemaphoreType.DMA,
        ],
    )
    return f(coords, table_flat)


# speedup vs baseline: 1.0139x; 1.0139x over previous
import functools

import jax
import jax.numpy as jnp
from jax import lax
from jax.experimental import pallas as pl
from jax.experimental.pallas import tpu as pltpu
from jax.experimental.pallas import tpu_sc as plsc

NUM_BINS = 8
INV_BIN = 8.0
NUM_HEADS = 16
SIDE = 2 * NUM_BINS + 1
TABLE_SIZE = SIDE ** 3
TPAD = 4992

NC, NS, L = 2, 16, 16
NW = NC * NS


def _splat(x):
    return jnp.broadcast_to(jnp.asarray(x, jnp.int32), (L,))


def _body(coords_hbm, table_hbm, out_hbm, coords_v, table_v, out_v,
          sem0, sem1, sem2, sem3, B, N):
    rows_per_w = (B * N) // NW
    tiles_per_b = N // rows_per_w
    wid = lax.axis_index("s") * NC + lax.axis_index("c")
    b = wid // tiles_per_b
    ibase = (wid % tiles_per_b) * rows_per_w

    pltpu.sync_copy(table_hbm, table_v)
    pltpu.sync_copy(coords_hbm.at[b], coords_v)

    sems = (sem0, sem1, sem2, sem3)
    n_chunks = N // L

    def splat_center(i):
        si = jnp.broadcast_to(i, (L,))
        return [plsc.load_gather(coords_v, [_splat(c), si]) + 8.5
                for c in range(3)]

    def do_pair(t2, _):
        r0 = t2 * 2
        iA, iB = ibase + r0, ibase + r0 + 1
        cA = splat_center(iA)
        cB = splat_center(iB)

        for half in range(2):
            j0 = half * (N // 2)
            mA, mB = half * 2, half * 2 + 1

            @pl.when(t2 > 0)
            def _():
                for m in (mA, mB):
                    pltpu.make_async_copy(
                        out_v.at[m], out_hbm.at[0, :, 0, pl.ds(0, N // 2)],
                        sems[m]).wait()

            @plsc.parallel_loop(0, n_chunks // 2, unroll=1)
            def chunk(jc):
                jb = jc * L
                cj = [coords_v[c, pl.ds(j0 + jb, L)] for c in range(3)]
                for (m, ci) in ((mA, cA), (mB, cB)):
                    q = [jnp.clip(ci[c] - cj[c], 0.01, 16.99)
                         for c in range(3)]
                    idx = (q[0].astype(jnp.int32) * (SIDE * SIDE)
                           + q[1].astype(jnp.int32) * SIDE
                           + q[2].astype(jnp.int32))
                    for h in range(NUM_HEADS):
                        val = plsc.load_gather(table_v, [_splat(h), idx])
                        out_v[m, h, pl.ds(jb, L)] = val

            pltpu.async_copy(
                out_v.at[mA], out_hbm.at[b, :, iA, pl.ds(j0, N // 2)],
                sems[mA])
            pltpu.async_copy(
                out_v.at[mB], out_hbm.at[b, :, iB, pl.ds(j0, N // 2)],
                sems[mB])
        return 0

    lax.fori_loop(0, rows_per_w // 2, do_pair, 0)
    for m in range(4):
        pltpu.make_async_copy(
            out_v.at[m], out_hbm.at[0, :, 0, pl.ds(0, N // 2)],
            sems[m]).wait()


def kernel(token_centers, bias_table):
    B, N, _ = token_centers.shape
    H = bias_table.shape[0]
    assert H == NUM_HEADS and bias_table.shape[1] == TABLE_SIZE
    assert (B * N) % (2 * NW) == 0 and N % (4 * L) == 0

    coords = jnp.transpose(token_centers, (0, 2, 1)) * INV_BIN

    mesh = plsc.VectorSubcoreMesh(
        core_axis_name="c", subcore_axis_name="s",
        num_cores=NC, num_subcores=NS)
    body = functools.partial(_body, B=B, N=N)
    f = pl.kernel(
        body,
        out_type=jax.ShapeDtypeStruct((B, H, N, N), jnp.float32),
        mesh=mesh,
        compiler_params=pltpu.CompilerParams(
            needs_layout_passes=False),
        scratch_types=[
            pltpu.VMEM((3, N), jnp.float32),
            pltpu.VMEM((H, TABLE_SIZE), jnp.float32),
            pltpu.VMEM((4, H, N // 2), jnp.float32),
            pltpu.SemaphoreType.DMA,
            pltpu.SemaphoreType.DMA,
            pltpu.SemaphoreType.DMA,
            pltpu.SemaphoreType.DMA,
        ],
    )
    return f(coords, bias_table)
